# Initial kernel scaffold; baseline (speedup 1.0000x reference)
import jax
import jax.numpy as jnp
from jax.experimental import pallas as pl


def kernel(x, edge_index, W, attn_l, attn_r):
    raise NotImplementedError("write your pallas kernel here")


# speedup vs baseline: 65.7582x; 65.7582x over previous
import jax
import jax.numpy as jnp
import numpy as np
from jax import lax
from jax.experimental import pallas as pl
from jax.experimental.pallas import tpu as pltpu
from jax.experimental.pallas import tpu_sc as plsc

N = 10000
E = 320000
IN_FEATS = 128
H = 8
F = 16
D = H * F

NC = 2
NS = 16
NW = NC * NS
EPW = E // NW
C = 80
NCHUNK = EPW // C

_BLK = 400
_NBLK = N // _BLK


def _prep_body(x_ref, wt_ref, al_ref, ar_ref, s2_ref, ft_ref, ell_ref, err_ref):
    ft = jnp.dot(x_ref[...], wt_ref[...], preferred_element_type=jnp.float32)
    ft_ref[...] = ft
    ell_ref[...] = jnp.dot(ft * al_ref[...], s2_ref[...],
                           preferred_element_type=jnp.float32)
    err_ref[...] = jnp.dot(ft * ar_ref[...], s2_ref[...],
                           preferred_element_type=jnp.float32)


def _edge_body(ft_hbm, ell_hbm, err_hbm, src_hbm, dst_hbm, zrow_hbm, zsum_hbm,
               outp_hbm, outs_hbm,
               rst_sh, esum_sh,
               srcc, dstc, ftb, elb, erb,
               sem0, sem1, sem2):
    c = lax.axis_index("c")
    s = lax.axis_index("s")
    wid = s * NC + c

    @pl.when(s == 0)
    def _():
        pltpu.sync_copy(zrow_hbm, rst_sh)
        pltpu.sync_copy(zsum_hbm, esum_sh)

    plsc.subcore_barrier()

    ebase = wid * EPW

    def chunk_body(k, carry):
        pltpu.sync_copy(src_hbm.at[pl.ds(ebase + k * C, C)], srcc)
        pltpu.sync_copy(dst_hbm.at[pl.ds(ebase + k * C, C)], dstc)
        cp0 = pltpu.async_copy(ft_hbm.at[srcc], ftb, sem0)
        cp1 = pltpu.async_copy(ell_hbm.at[srcc], elb, sem1)
        cp2 = pltpu.async_copy(err_hbm.at[dstc], erb, sem2)
        cp1.wait()
        cp2.wait()

        def w_body(j, carry2):
            e16 = elb[j, :] + erb[j, :]
            e16 = jnp.where(e16 >= 0.0, e16, 0.2 * e16)
            elb[j, :] = jnp.exp(e16)
            return carry2

        lax.fori_loop(0, C, w_body, 0, unroll=2)
        cp0.wait()

        def m_body(j, carry2):
            wv = elb[j, :]
            for h in range(H):
                ftb[j, pl.ds(h * F, F)] = ftb[j, pl.ds(h * F, F)] * wv[h]
            return carry2

        lax.fori_loop(0, C, m_body, 0)

        pltpu.sync_copy(ftb, rst_sh.at[dstc], add=True)
        pltpu.sync_copy(elb, esum_sh.at[dstc], add=True)
        return carry

    lax.fori_loop(0, NCHUNK, chunk_body, 0)

    plsc.subcore_barrier()

    @pl.when(s == 0)
    def _():
        pltpu.sync_copy(rst_sh, outp_hbm.at[pl.ds(c * N, N)])
        pltpu.sync_copy(esum_sh, outs_hbm.at[pl.ds(c * N, N)])


def _combine_body(p_ref, s_ref, b16_ref, out_ref):
    ps = p_ref[0] + p_ref[1]
    ss = s_ref[0] + s_ref[1]
    inv = jnp.where(ss > 0.0, 1.0 / ss, 0.0)
    expand = jnp.dot(inv, b16_ref[...], preferred_element_type=jnp.float32)
    out_ref[...] = ps * expand


@jax.jit
def _gat(x, src, dst, wt, al, ar):
    s2 = np.zeros((D, F), dtype=np.float32)
    for h in range(H):
        s2[h * F:(h + 1) * F, h] = 1.0
    b16 = np.zeros((F, D), dtype=np.float32)
    for h in range(H):
        b16[h, h * F:(h + 1) * F] = 1.0
    s2 = jnp.asarray(s2)
    b16 = jnp.asarray(b16)

    ft, ell, err = pl.pallas_call(
        _prep_body,
        grid=(_NBLK,),
        in_specs=[
            pl.BlockSpec((_BLK, IN_FEATS), lambda i: (i, 0)),
            pl.BlockSpec((IN_FEATS, D), lambda i: (0, 0)),
            pl.BlockSpec((1, D), lambda i: (0, 0)),
            pl.BlockSpec((1, D), lambda i: (0, 0)),
            pl.BlockSpec((D, F), lambda i: (0, 0)),
        ],
        out_specs=[
            pl.BlockSpec((_BLK, D), lambda i: (i, 0)),
            pl.BlockSpec((_BLK, F), lambda i: (i, 0)),
            pl.BlockSpec((_BLK, F), lambda i: (i, 0)),
        ],
        out_shape=[
            jax.ShapeDtypeStruct((N, D), jnp.float32),
            jax.ShapeDtypeStruct((N, F), jnp.float32),
            jax.ShapeDtypeStruct((N, F), jnp.float32),
        ],
    )(x, wt, al, ar, s2)

    zrow = jnp.zeros((N, D), jnp.float32)
    zsum = jnp.zeros((N, F), jnp.float32)

    edge_kernel = pl.kernel(
        _edge_body,
        out_type=[
            jax.ShapeDtypeStruct((NC * N, D), jnp.float32),
            jax.ShapeDtypeStruct((NC * N, F), jnp.float32),
        ],
        mesh=plsc.VectorSubcoreMesh(core_axis_name="c", subcore_axis_name="s"),
        compiler_params=pltpu.CompilerParams(use_tc_tiling_on_sc=False),
        scratch_types=[
            pltpu.VMEM_SHARED((N, D), jnp.float32),
            pltpu.VMEM_SHARED((N, F), jnp.float32),
            pltpu.VMEM((C,), jnp.int32),
            pltpu.VMEM((C,), jnp.int32),
            pltpu.VMEM((C, D), jnp.float32),
            pltpu.VMEM((C, F), jnp.float32),
            pltpu.VMEM((C, F), jnp.float32),
            pltpu.SemaphoreType.DMA,
            pltpu.SemaphoreType.DMA,
            pltpu.SemaphoreType.DMA,
        ],
    )
    outp, outs = edge_kernel(ft, ell, err, src, dst, zrow, zsum)

    outp = outp.reshape(NC, N, D)
    outs = outs.reshape(NC, N, F)

    out = pl.pallas_call(
        _combine_body,
        grid=(_NBLK,),
        in_specs=[
            pl.BlockSpec((NC, _BLK, D), lambda i: (0, i, 0)),
            pl.BlockSpec((NC, _BLK, F), lambda i: (0, i, 0)),
            pl.BlockSpec((F, D), lambda i: (0, 0)),
        ],
        out_specs=pl.BlockSpec((_BLK, D), lambda i: (i, 0)),
        out_shape=jax.ShapeDtypeStruct((N, D), jnp.float32),
    )(outp, outs, b16)

    return out.reshape(N, H, F)


def kernel(x, edge_index, W, attn_l, attn_r):
    src = edge_index[0].astype(jnp.int32)
    dst = edge_index[1].astype(jnp.int32)
    wt = W.T
    al = attn_l.reshape(1, D)
    ar = attn_r.reshape(1, D)
    return _gat(x, src, dst, wt, al, ar)


# speedup vs baseline: 99.7827x; 1.5174x over previous
import jax
import jax.numpy as jnp
import numpy as np
from jax import lax
from jax.experimental import pallas as pl
from jax.experimental.pallas import tpu as pltpu
from jax.experimental.pallas import tpu_sc as plsc

N = 10000
E = 320000
IN_FEATS = 128
H = 8
F = 16
D = H * F

NC = 2
NS = 16
NW = NC * NS
EPW = E // NW
C = 40
NCHUNK = EPW // C
RB = 3
NITER = NCHUNK // RB
NTAIL = NCHUNK % RB

_BLK = 400
_NBLK = N // _BLK


def _prep_body(x_ref, wt_ref, al_ref, ar_ref, s2_ref, ft_ref, ell_ref, err_ref):
    ft = jnp.dot(x_ref[...], wt_ref[...], preferred_element_type=jnp.float32)
    ft_ref[...] = ft
    ell_ref[...] = jnp.dot(ft * al_ref[...], s2_ref[...],
                           preferred_element_type=jnp.float32)
    err_ref[...] = jnp.dot(ft * ar_ref[...], s2_ref[...],
                           preferred_element_type=jnp.float32)


def _edge_body(ft_hbm, ell_hbm, err_hbm, src_hbm, dst_hbm, zrow_hbm, zsum_hbm,
               outp_hbm, outs_hbm,
               rst_sh, esum_sh, srcv, dstv, ftb, elb, erb,
               g0, g1, g2, s0, s1, s2):
    gsems = (g0, g1, g2)
    ssems = (s0, s1, s2)
    c = lax.axis_index("c")
    s = lax.axis_index("s")
    wid = s * NC + c

    @pl.when(s == 0)
    def _():
        pltpu.sync_copy(zrow_hbm, rst_sh)
        pltpu.sync_copy(zsum_hbm, esum_sh)

    pltpu.sync_copy(src_hbm.at[wid], srcv)
    pltpu.sync_copy(dst_hbm.at[wid], dstv)

    plsc.subcore_barrier()

    def start_gathers(k, b):
        pltpu.async_copy(ft_hbm.at[srcv.at[k]], ftb.at[b], gsems[b])
        pltpu.async_copy(ell_hbm.at[srcv.at[k]], elb.at[b], gsems[b])
        pltpu.async_copy(err_hbm.at[dstv.at[k]], erb.at[b], gsems[b])

    def drain_gathers(k, b):
        pltpu.make_async_copy(ft_hbm.at[srcv.at[k]], ftb.at[b], gsems[b]).wait()
        pltpu.make_async_copy(ell_hbm.at[srcv.at[k]], elb.at[b], gsems[b]).wait()
        pltpu.make_async_copy(err_hbm.at[dstv.at[k]], erb.at[b], gsems[b]).wait()

    def start_scatters(k, b):
        pltpu.async_copy(ftb.at[b], rst_sh.at[dstv.at[k]], ssems[b], add=True)
        pltpu.async_copy(elb.at[b], esum_sh.at[dstv.at[k]], ssems[b], add=True)

    def drain_scatters(k, b):
        pltpu.make_async_copy(ftb.at[b], rst_sh.at[dstv.at[k]], ssems[b]).wait()
        pltpu.make_async_copy(elb.at[b], esum_sh.at[dstv.at[k]], ssems[b]).wait()

    def compute(b):
        def body(j, carry):
            e16 = elb[b, j, :] + erb[b, j, :]
            e16 = jnp.where(e16 >= 0.0, e16, 0.2 * e16)
            w = jnp.exp(e16)
            elb[b, j, :] = w
            for h in range(H):
                ftb[b, j, pl.ds(h * F, F)] = ftb[b, j, pl.ds(h * F, F)] * w[h]
            return carry

        lax.fori_loop(0, C, body, 0)

    for b in range(RB - 1):
        start_gathers(b, b)

    def outer(g, carry):
        base = g * RB
        for b in range(RB):
            k = base + b
            kn = k + RB - 1
            bn = (b + RB - 1) % RB

            @pl.when(kn < NCHUNK)
            def _(kn=kn, bn=bn):
                @pl.when(kn >= RB)
                def _():
                    drain_scatters(kn - RB, bn)

                start_gathers(kn, bn)

            drain_gathers(k, b)
            compute(b)
            start_scatters(k, b)
        return carry

    lax.fori_loop(0, NITER, outer, 0)

    for t in range(NCHUNK - NTAIL, NCHUNK):
        b = t % RB
        drain_gathers(t, b)
        compute(b)
        start_scatters(t, b)

    for t in range(NCHUNK - RB, NCHUNK):
        drain_scatters(t, t % RB)

    plsc.subcore_barrier()

    @pl.when(s == 0)
    def _():
        pltpu.sync_copy(rst_sh, outp_hbm.at[pl.ds(c * N, N)])
        pltpu.sync_copy(esum_sh, outs_hbm.at[pl.ds(c * N, N)])


def _combine_body(p_ref, s_ref, b16_ref, out_ref):
    ps = p_ref[0] + p_ref[1]
    ss = s_ref[0] + s_ref[1]
    inv = jnp.where(ss > 0.0, 1.0 / ss, 0.0)
    expand = jnp.dot(inv, b16_ref[...], preferred_element_type=jnp.float32)
    out_ref[...] = ps * expand


@jax.jit
def _gat(x, src, dst, wt, al, ar):
    s2 = np.zeros((D, F), dtype=np.float32)
    for h in range(H):
        s2[h * F:(h + 1) * F, h] = 1.0
    b16 = np.zeros((F, D), dtype=np.float32)
    for h in range(H):
        b16[h, h * F:(h + 1) * F] = 1.0
    s2 = jnp.asarray(s2)
    b16 = jnp.asarray(b16)

    ft, ell, err = pl.pallas_call(
        _prep_body,
        grid=(_NBLK,),
        in_specs=[
            pl.BlockSpec((_BLK, IN_FEATS), lambda i: (i, 0)),
            pl.BlockSpec((IN_FEATS, D), lambda i: (0, 0)),
            pl.BlockSpec((1, D), lambda i: (0, 0)),
            pl.BlockSpec((1, D), lambda i: (0, 0)),
            pl.BlockSpec((D, F), lambda i: (0, 0)),
        ],
        out_specs=[
            pl.BlockSpec((_BLK, D), lambda i: (i, 0)),
            pl.BlockSpec((_BLK, F), lambda i: (i, 0)),
            pl.BlockSpec((_BLK, F), lambda i: (i, 0)),
        ],
        out_shape=[
            jax.ShapeDtypeStruct((N, D), jnp.float32),
            jax.ShapeDtypeStruct((N, F), jnp.float32),
            jax.ShapeDtypeStruct((N, F), jnp.float32),
        ],
    )(x, wt, al, ar, s2)

    zrow = jnp.zeros((N, D), jnp.float32)
    zsum = jnp.zeros((N, F), jnp.float32)

    edge_kernel = pl.kernel(
        _edge_body,
        out_type=[
            jax.ShapeDtypeStruct((NC * N, D), jnp.float32),
            jax.ShapeDtypeStruct((NC * N, F), jnp.float32),
        ],
        mesh=plsc.VectorSubcoreMesh(core_axis_name="c", subcore_axis_name="s"),
        compiler_params=pltpu.CompilerParams(use_tc_tiling_on_sc=False),
        scratch_types=[
            pltpu.VMEM_SHARED((N, D), jnp.float32),
            pltpu.VMEM_SHARED((N, F), jnp.float32),
            pltpu.VMEM((NCHUNK, C), jnp.int32),
            pltpu.VMEM((NCHUNK, C), jnp.int32),
            pltpu.VMEM((RB, C, D), jnp.float32),
            pltpu.VMEM((RB, C, F), jnp.float32),
            pltpu.VMEM((RB, C, F), jnp.float32),
            pltpu.SemaphoreType.DMA,
            pltpu.SemaphoreType.DMA,
            pltpu.SemaphoreType.DMA,
            pltpu.SemaphoreType.DMA,
            pltpu.SemaphoreType.DMA,
            pltpu.SemaphoreType.DMA,
        ],
    )
    outp, outs = edge_kernel(ft, ell, err, src, dst, zrow, zsum)

    outp = outp.reshape(NC, N, D)
    outs = outs.reshape(NC, N, F)

    out = pl.pallas_call(
        _combine_body,
        grid=(_NBLK,),
        in_specs=[
            pl.BlockSpec((NC, _BLK, D), lambda i: (0, i, 0)),
            pl.BlockSpec((NC, _BLK, F), lambda i: (0, i, 0)),
            pl.BlockSpec((F, D), lambda i: (0, 0)),
        ],
        out_specs=pl.BlockSpec((_BLK, D), lambda i: (i, 0)),
        out_shape=jax.ShapeDtypeStruct((N, D), jnp.float32),
    )(outp, outs, b16)

    return out.reshape(N, H, F)


def kernel(x, edge_index, W, attn_l, attn_r):
    src = edge_index[0].astype(jnp.int32).reshape(NW, NCHUNK, C)
    dst = edge_index[1].astype(jnp.int32).reshape(NW, NCHUNK, C)
    wt = W.T
    al = attn_l.reshape(1, D)
    ar = attn_r.reshape(1, D)
    return _gat(x, src, dst, wt, al, ar)


# speedup vs baseline: 162.0290x; 1.6238x over previous
import jax
import jax.numpy as jnp
import numpy as np
from jax import lax
from jax.experimental import pallas as pl
from jax.experimental.pallas import tpu as pltpu
from jax.experimental.pallas import tpu_sc as plsc

N = 10000
E = 320000
IN_FEATS = 128
H = 8
F = 16
D = H * F

NC = 2
NS = 16
NW = NC * NS
EPW = E // NW
C = 40
NCHUNK = EPW // C
RB = 3
NITER = NCHUNK // RB
NTAIL = NCHUNK % RB

_BLK = 400
_NBLK = N // _BLK


def _prep_body(x_ref, wt_ref, al_ref, ar_ref, s2_ref, ft_ref, ell_ref, err_ref):
    ft = jnp.dot(x_ref[...], wt_ref[...], preferred_element_type=jnp.float32)
    ft_ref[...] = ft
    ell_ref[...] = jnp.dot(ft * al_ref[...], s2_ref[...],
                           preferred_element_type=jnp.float32)
    err_ref[...] = jnp.dot(ft * ar_ref[...], s2_ref[...],
                           preferred_element_type=jnp.float32)


def _edge_body(ft_hbm, ell_hbm, err_hbm, src_hbm, dst_hbm, zrow_hbm, zsum_hbm,
               outp_hbm, outs_hbm,
               rst_sh, esum_sh, srcv, dstv, ftb, elb, erb,
               g0, g1, g2, s0, s1, s2):
    gsems = (g0, g1, g2)
    ssems = (s0, s1, s2)
    c = lax.axis_index("c")
    s = lax.axis_index("s")
    wid = s * NC + c

    rbase = s * (N // NS)
    pltpu.sync_copy(zrow_hbm.at[pl.ds(rbase, N // NS)],
                    rst_sh.at[pl.ds(rbase, N // NS)])
    pltpu.sync_copy(zsum_hbm.at[pl.ds(rbase, N // NS)],
                    esum_sh.at[pl.ds(rbase, N // NS)])

    pltpu.sync_copy(src_hbm.at[wid], srcv)
    pltpu.sync_copy(dst_hbm.at[wid], dstv)

    plsc.subcore_barrier()

    def start_gathers(k, b):
        pltpu.async_copy(ft_hbm.at[srcv.at[k]], ftb.at[b], gsems[b])
        pltpu.async_copy(ell_hbm.at[srcv.at[k]], elb.at[b], gsems[b])
        pltpu.async_copy(err_hbm.at[dstv.at[k]], erb.at[b], gsems[b])

    def drain_gathers(k, b):
        pltpu.make_async_copy(ft_hbm.at[srcv.at[k]], ftb.at[b], gsems[b]).wait()
        pltpu.make_async_copy(ell_hbm.at[srcv.at[k]], elb.at[b], gsems[b]).wait()
        pltpu.make_async_copy(err_hbm.at[dstv.at[k]], erb.at[b], gsems[b]).wait()

    def start_scatters(k, b):
        pltpu.async_copy(ftb.at[b], rst_sh.at[dstv.at[k]], ssems[b], add=True)
        pltpu.async_copy(elb.at[b], esum_sh.at[dstv.at[k]], ssems[b], add=True)

    def drain_scatters(k, b):
        pltpu.make_async_copy(ftb.at[b], rst_sh.at[dstv.at[k]], ssems[b]).wait()
        pltpu.make_async_copy(elb.at[b], esum_sh.at[dstv.at[k]], ssems[b]).wait()

    def compute(b):
        @plsc.parallel_loop(0, C, unroll=2)
        def _(j):
            e16 = elb[b, j, :] + erb[b, j, :]
            e16 = jnp.where(e16 >= 0.0, e16, 0.2 * e16)
            w = jnp.exp(e16)
            elb[b, j, :] = w
            for h in range(H):
                ftb[b, j, pl.ds(h * F, F)] = ftb[b, j, pl.ds(h * F, F)] * w[h]

    for b in range(RB - 1):
        start_gathers(b, b)

    def outer(g, carry):
        base = g * RB
        for b in range(RB):
            k = base + b
            kn = k + RB - 1
            bn = (b + RB - 1) % RB

            @pl.when(kn < NCHUNK)
            def _(kn=kn, bn=bn):
                @pl.when(kn >= RB)
                def _():
                    drain_scatters(kn - RB, bn)

                start_gathers(kn, bn)

            drain_gathers(k, b)
            compute(b)
            start_scatters(k, b)
        return carry

    lax.fori_loop(0, NITER, outer, 0)

    for t in range(NCHUNK - NTAIL, NCHUNK):
        b = t % RB
        drain_gathers(t, b)
        compute(b)
        start_scatters(t, b)

    for t in range(NCHUNK - RB, NCHUNK):
        drain_scatters(t, t % RB)

    plsc.subcore_barrier()

    pltpu.sync_copy(rst_sh.at[pl.ds(rbase, N // NS)],
                    outp_hbm.at[pl.ds(c * N + rbase, N // NS)])
    pltpu.sync_copy(esum_sh.at[pl.ds(rbase, N // NS)],
                    outs_hbm.at[pl.ds(c * N + rbase, N // NS)])


def _combine_body(p_ref, s_ref, b16_ref, out_ref):
    ps = p_ref[0] + p_ref[1]
    ss = s_ref[0] + s_ref[1]
    inv = jnp.where(ss > 0.0, 1.0 / ss, 0.0)
    expand = jnp.dot(inv, b16_ref[...], preferred_element_type=jnp.float32)
    out_ref[...] = ps * expand


@jax.jit
def _gat(x, src, dst, wt, al, ar):
    s2 = np.zeros((D, F), dtype=np.float32)
    for h in range(H):
        s2[h * F:(h + 1) * F, h] = 1.0
    b16 = np.zeros((F, D), dtype=np.float32)
    for h in range(H):
        b16[h, h * F:(h + 1) * F] = 1.0
    s2 = jnp.asarray(s2)
    b16 = jnp.asarray(b16)

    ft, ell, err = pl.pallas_call(
        _prep_body,
        grid=(_NBLK,),
        in_specs=[
            pl.BlockSpec((_BLK, IN_FEATS), lambda i: (i, 0)),
            pl.BlockSpec((IN_FEATS, D), lambda i: (0, 0)),
            pl.BlockSpec((1, D), lambda i: (0, 0)),
            pl.BlockSpec((1, D), lambda i: (0, 0)),
            pl.BlockSpec((D, F), lambda i: (0, 0)),
        ],
        out_specs=[
            pl.BlockSpec((_BLK, D), lambda i: (i, 0)),
            pl.BlockSpec((_BLK, F), lambda i: (i, 0)),
            pl.BlockSpec((_BLK, F), lambda i: (i, 0)),
        ],
        out_shape=[
            jax.ShapeDtypeStruct((N, D), jnp.float32),
            jax.ShapeDtypeStruct((N, F), jnp.float32),
            jax.ShapeDtypeStruct((N, F), jnp.float32),
        ],
    )(x, wt, al, ar, s2)

    zrow = jnp.zeros((N, D), jnp.float32)
    zsum = jnp.zeros((N, F), jnp.float32)

    edge_kernel = pl.kernel(
        _edge_body,
        out_type=[
            jax.ShapeDtypeStruct((NC * N, D), jnp.float32),
            jax.ShapeDtypeStruct((NC * N, F), jnp.float32),
        ],
        mesh=plsc.VectorSubcoreMesh(core_axis_name="c", subcore_axis_name="s"),
        compiler_params=pltpu.CompilerParams(use_tc_tiling_on_sc=False),
        scratch_types=[
            pltpu.VMEM_SHARED((N, D), jnp.float32),
            pltpu.VMEM_SHARED((N, F), jnp.float32),
            pltpu.VMEM((NCHUNK, C), jnp.int32),
            pltpu.VMEM((NCHUNK, C), jnp.int32),
            pltpu.VMEM((RB, C, D), jnp.float32),
            pltpu.VMEM((RB, C, F), jnp.float32),
            pltpu.VMEM((RB, C, F), jnp.float32),
            pltpu.SemaphoreType.DMA,
            pltpu.SemaphoreType.DMA,
            pltpu.SemaphoreType.DMA,
            pltpu.SemaphoreType.DMA,
            pltpu.SemaphoreType.DMA,
            pltpu.SemaphoreType.DMA,
        ],
    )
    outp, outs = edge_kernel(ft, ell, err, src, dst, zrow, zsum)

    outp = outp.reshape(NC, N, D)
    outs = outs.reshape(NC, N, F)

    out = pl.pallas_call(
        _combine_body,
        grid=(_NBLK,),
        in_specs=[
            pl.BlockSpec((NC, _BLK, D), lambda i: (0, i, 0)),
            pl.BlockSpec((NC, _BLK, F), lambda i: (0, i, 0)),
            pl.BlockSpec((F, D), lambda i: (0, 0)),
        ],
        out_specs=pl.BlockSpec((_BLK, D), lambda i: (i, 0)),
        out_shape=jax.ShapeDtypeStruct((N, D), jnp.float32),
    )(outp, outs, b16)

    return out.reshape(N, H, F)


def kernel(x, edge_index, W, attn_l, attn_r):
    src = edge_index[0].astype(jnp.int32).reshape(NW, NCHUNK, C)
    dst = edge_index[1].astype(jnp.int32).reshape(NW, NCHUNK, C)
    wt = W.T
    al = attn_l.reshape(1, D)
    ar = attn_r.reshape(1, D)
    return _gat(x, src, dst, wt, al, ar)
